# baseline (speedup 1.0000x reference)
import functools

import jax
import jax.numpy as jnp
from jax import lax
from jax.experimental import pallas as pl
from jax.experimental.pallas import tpu as pltpu
from jax.experimental.pallas import tpu_sc as plsc

TILE_N = 1024
TILE_K = 4096
N_TOK = 16384
K_CB = 8192
DIM = 32


def _argmin_body(xb2_ref, ct_ref, xsq_ref, csq_ref, oidx_ref,
                 mmin_ref, midx_ref):
    j = pl.program_id(1)

    @pl.when(j == 0)
    def _init():
        mmin_ref[...] = jnp.full_like(mmin_ref, jnp.inf)
        midx_ref[...] = jnp.zeros_like(midx_ref)

    m2 = jax.lax.dot_general(
        xb2_ref[...], ct_ref[...], (((1,), (0,)), ((), ())),
        preferred_element_type=jnp.float32)
    d = xsq_ref[...] - m2 + csq_ref[...]
    bmin = jnp.min(d, axis=1, keepdims=True)
    bidx = (jnp.argmin(d, axis=1).astype(jnp.int32).reshape(-1, 1)
            + j * TILE_K)
    better = bmin < mmin_ref[...]
    midx_ref[...] = jnp.where(better, bidx, midx_ref[...])
    newmin = jnp.where(better, bmin, mmin_ref[...])
    mmin_ref[...] = newmin.astype(jnp.bfloat16).astype(jnp.float32)

    @pl.when(j == pl.num_programs(1) - 1)
    def _flush():
        oidx_ref[...] = midx_ref[...]


@jax.jit
def _argmin_indices(xb2, ctb, xsq, csq):
    grid = (N_TOK // TILE_N, K_CB // TILE_K)
    return pl.pallas_call(
        _argmin_body,
        grid=grid,
        in_specs=[
            pl.BlockSpec((TILE_N, DIM), lambda i, j: (i, 0)),
            pl.BlockSpec((DIM, TILE_K), lambda i, j: (0, j)),
            pl.BlockSpec((TILE_N, 1), lambda i, j: (i, 0)),
            pl.BlockSpec((1, TILE_K), lambda i, j: (0, j)),
        ],
        out_specs=pl.BlockSpec((TILE_N, 1), lambda i, j: (i, 0)),
        out_shape=jax.ShapeDtypeStruct((N_TOK, 1), jnp.int32),
        scratch_shapes=[
            pltpu.VMEM((TILE_N, 1), jnp.float32),
            pltpu.VMEM((TILE_N, 1), jnp.int32),
        ],
        compiler_params=pltpu.CompilerParams(
            dimension_semantics=("parallel", "arbitrary")),
    )(xb2, ctb, xsq, csq)


def _seq_rowsum_sq(a):
    e = a * a
    acc = e[:, 0]
    for k in range(1, a.shape[1]):
        acc = acc + e[:, k]
    return acc


GD = 128


@functools.cache
def _gather_kernel():
    info = plsc.get_sparse_core_info()
    nc, ns = info.num_cores, info.num_subcores
    nw = nc * ns
    b_per_w = N_TOK // nw
    mesh = plsc.VectorSubcoreMesh(core_axis_name="c", subcore_axis_name="s")

    @functools.partial(
        pl.kernel, mesh=mesh,
        out_type=jax.ShapeDtypeStruct((N_TOK, GD), jnp.float32),
        scratch_types=[
            pltpu.VMEM((b_per_w,), jnp.int32),
            pltpu.VMEM((b_per_w, GD), jnp.float32),
            pltpu.SemaphoreType.DMA,
        ],
    )
    def gather(table_hbm, idx_hbm, out_hbm, idx_v, rows_v, sem):
        wid = lax.axis_index("s") * nc + lax.axis_index("c")
        base = wid * b_per_w
        pltpu.sync_copy(idx_hbm.at[pl.ds(base, b_per_w)], idx_v)
        pltpu.async_copy(table_hbm.at[idx_v], rows_v, sem).wait()
        pltpu.sync_copy(rows_v, out_hbm.at[pl.ds(base, b_per_w)])

    return gather


def kernel(inputs, codebook):
    B, C, H, W = inputs.shape
    x = jnp.transpose(inputs, (0, 2, 3, 1)).reshape(-1, C)
    xb2 = (2.0 * x).astype(jnp.bfloat16)
    ctf = codebook.T
    ctb = ctf.astype(jnp.bfloat16)
    xsq = _seq_rowsum_sq(x)[:, None]
    csq = _seq_rowsum_sq(codebook)[None, :]
    idx = (xsq.reshape(-1) > 32.0).astype(jnp.int32)
    table = jnp.pad(codebook, ((0, 0), (0, GD - C)))
    q = _gather_kernel()(table, idx)[:, :C]
    ste = x + jax.lax.stop_gradient(q - x)
    return jnp.transpose(ste.reshape(B, H, W, C), (0, 3, 1, 2))


# speedup vs baseline: 1.5359x; 1.5359x over previous
import functools

import jax
import jax.numpy as jnp
from jax import lax
from jax.experimental import pallas as pl
from jax.experimental.pallas import tpu as pltpu
from jax.experimental.pallas import tpu_sc as plsc

TILE_N = 1024
TILE_K = 4096
N_TOK = 16384
K_CB = 8192
DIM = 32


def _argmin_body(xb2_ref, ct_ref, x_ref, ctf_ref, oidx_ref,
                 mmin_ref, midx_ref, xsq_ref):
    j = pl.program_id(1)

    @pl.when(j == 0)
    def _init():
        mmin_ref[...] = jnp.full_like(mmin_ref, jnp.inf)
        midx_ref[...] = jnp.zeros_like(midx_ref)
        x = x_ref[...]
        e = x * x
        acc = e[:, 0:1]
        for k in range(1, DIM):
            acc = acc + e[:, k:k + 1]
        xsq_ref[...] = acc

    ctf = ctf_ref[...]
    ec = ctf * ctf
    csq = ec[0:1, :]
    for c in range(1, DIM):
        csq = csq + ec[c:c + 1, :]

    m2 = jax.lax.dot_general(
        xb2_ref[...], ct_ref[...], (((1,), (0,)), ((), ())),
        preferred_element_type=jnp.float32)
    d = xsq_ref[...] - m2 + csq
    bmin = jnp.min(d, axis=1, keepdims=True)
    bidx = (jnp.argmin(d, axis=1).astype(jnp.int32).reshape(-1, 1)
            + j * TILE_K)
    better = bmin < mmin_ref[...]
    midx_ref[...] = jnp.where(better, bidx, midx_ref[...])
    newmin = jnp.where(better, bmin, mmin_ref[...])
    mmin_ref[...] = newmin.astype(jnp.bfloat16).astype(jnp.float32)

    @pl.when(j == pl.num_programs(1) - 1)
    def _flush():
        oidx_ref[...] = midx_ref[...]


@jax.jit
def _argmin_indices(xb2, ctb, x, ctf):
    grid = (N_TOK // TILE_N, K_CB // TILE_K)
    return pl.pallas_call(
        _argmin_body,
        grid=grid,
        in_specs=[
            pl.BlockSpec((TILE_N, DIM), lambda i, j: (i, 0)),
            pl.BlockSpec((DIM, TILE_K), lambda i, j: (0, j)),
            pl.BlockSpec((TILE_N, DIM), lambda i, j: (i, 0)),
            pl.BlockSpec((DIM, TILE_K), lambda i, j: (0, j)),
        ],
        out_specs=pl.BlockSpec((TILE_N, 1), lambda i, j: (i, 0)),
        out_shape=jax.ShapeDtypeStruct((N_TOK, 1), jnp.int32),
        scratch_shapes=[
            pltpu.VMEM((TILE_N, 1), jnp.float32),
            pltpu.VMEM((TILE_N, 1), jnp.int32),
            pltpu.VMEM((TILE_N, 1), jnp.float32),
        ],
        compiler_params=pltpu.CompilerParams(
            dimension_semantics=("parallel", "arbitrary")),
    )(xb2, ctb, x, ctf)


GD = 128


@functools.cache
def _gather_kernel():
    info = plsc.get_sparse_core_info()
    nc, ns = info.num_cores, info.num_subcores
    nw = nc * ns
    b_per_w = N_TOK // nw
    mesh = plsc.VectorSubcoreMesh(core_axis_name="c", subcore_axis_name="s")

    @functools.partial(
        pl.kernel, mesh=mesh,
        out_type=jax.ShapeDtypeStruct((N_TOK, GD), jnp.float32),
        scratch_types=[
            pltpu.VMEM((b_per_w,), jnp.int32),
            pltpu.VMEM((b_per_w, GD), jnp.float32),
            pltpu.SemaphoreType.DMA,
        ],
    )
    def gather(table_hbm, idx_hbm, out_hbm, idx_v, rows_v, sem):
        wid = lax.axis_index("s") * nc + lax.axis_index("c")
        base = wid * b_per_w
        pltpu.sync_copy(idx_hbm.at[pl.ds(base, b_per_w)], idx_v)
        pltpu.async_copy(table_hbm.at[idx_v], rows_v, sem).wait()
        pltpu.sync_copy(rows_v, out_hbm.at[pl.ds(base, b_per_w)])

    return gather


def kernel(inputs, codebook):
    B, C, H, W = inputs.shape
    x = jnp.transpose(inputs, (0, 2, 3, 1)).reshape(-1, C)
    xb2 = (2.0 * x).astype(jnp.bfloat16)
    ctf = codebook.T
    ctb = ctf.astype(jnp.bfloat16)
    idx = _argmin_indices(xb2, ctb, x, ctf).reshape(-1)
    table = jnp.pad(codebook, ((0, 0), (0, GD - C)))
    q = _gather_kernel()(table, idx)[:, :C]
    ste = x + jax.lax.stop_gradient(q - x)
    return jnp.transpose(ste.reshape(B, H, W, C), (0, 3, 1, 2))


# speedup vs baseline: 1.5699x; 1.0221x over previous
import functools

import jax
import jax.numpy as jnp
from jax import lax
from jax.experimental import pallas as pl
from jax.experimental.pallas import tpu as pltpu
from jax.experimental.pallas import tpu_sc as plsc

TILE_N = 2048
TILE_K = 4096
N_TOK = 16384
K_CB = 8192
DIM = 32


def _argmin_body(xb2_ref, ct_ref, x_ref, ctf_ref, oidx_ref,
                 mmin_ref, midx_ref, xsq_ref):
    j = pl.program_id(1)

    @pl.when(j == 0)
    def _init():
        mmin_ref[...] = jnp.full_like(mmin_ref, jnp.inf)
        midx_ref[...] = jnp.zeros_like(midx_ref)
        x = x_ref[...]
        e = x * x
        acc = e[:, 0:1]
        for k in range(1, DIM):
            acc = acc + e[:, k:k + 1]
        xsq_ref[...] = acc

    ctf = ctf_ref[...]
    ec = ctf * ctf
    csq = ec[0:1, :]
    for c in range(1, DIM):
        csq = csq + ec[c:c + 1, :]

    m2 = jax.lax.dot_general(
        xb2_ref[...], ct_ref[...], (((1,), (0,)), ((), ())),
        preferred_element_type=jnp.float32)
    d = xsq_ref[...] - m2 + csq
    bmin = jnp.min(d, axis=1, keepdims=True)
    bidx = (jnp.argmin(d, axis=1).astype(jnp.int32).reshape(-1, 1)
            + j * TILE_K)
    better = bmin < mmin_ref[...]
    midx_ref[...] = jnp.where(better, bidx, midx_ref[...])
    newmin = jnp.where(better, bmin, mmin_ref[...])
    mmin_ref[...] = newmin.astype(jnp.bfloat16).astype(jnp.float32)

    @pl.when(j == pl.num_programs(1) - 1)
    def _flush():
        oidx_ref[...] = midx_ref[...]


@jax.jit
def _argmin_indices(xb2, ctb, x, ctf):
    grid = (N_TOK // TILE_N, K_CB // TILE_K)
    return pl.pallas_call(
        _argmin_body,
        grid=grid,
        in_specs=[
            pl.BlockSpec((TILE_N, DIM), lambda i, j: (i, 0)),
            pl.BlockSpec((DIM, TILE_K), lambda i, j: (0, j)),
            pl.BlockSpec((TILE_N, DIM), lambda i, j: (i, 0)),
            pl.BlockSpec((DIM, TILE_K), lambda i, j: (0, j)),
        ],
        out_specs=pl.BlockSpec((TILE_N, 1), lambda i, j: (i, 0)),
        out_shape=jax.ShapeDtypeStruct((N_TOK, 1), jnp.int32),
        scratch_shapes=[
            pltpu.VMEM((TILE_N, 1), jnp.float32),
            pltpu.VMEM((TILE_N, 1), jnp.int32),
            pltpu.VMEM((TILE_N, 1), jnp.float32),
        ],
        compiler_params=pltpu.CompilerParams(
            dimension_semantics=("parallel", "arbitrary")),
    )(xb2, ctb, x, ctf)


GD = 128


@functools.cache
def _gather_kernel():
    info = plsc.get_sparse_core_info()
    nc, ns = info.num_cores, info.num_subcores
    nw = nc * ns
    b_per_w = N_TOK // nw
    mesh = plsc.VectorSubcoreMesh(core_axis_name="c", subcore_axis_name="s")

    @functools.partial(
        pl.kernel, mesh=mesh,
        out_type=jax.ShapeDtypeStruct((N_TOK, GD), jnp.float32),
        scratch_types=[
            pltpu.VMEM((b_per_w,), jnp.int32),
            pltpu.VMEM((b_per_w, GD), jnp.float32),
            pltpu.SemaphoreType.DMA,
        ],
    )
    def gather(table_hbm, idx_hbm, out_hbm, idx_v, rows_v, sem):
        wid = lax.axis_index("s") * nc + lax.axis_index("c")
        base = wid * b_per_w
        pltpu.sync_copy(idx_hbm.at[pl.ds(base, b_per_w)], idx_v)
        pltpu.async_copy(table_hbm.at[idx_v], rows_v, sem).wait()
        pltpu.sync_copy(rows_v, out_hbm.at[pl.ds(base, b_per_w)])

    return gather


def kernel(inputs, codebook):
    B, C, H, W = inputs.shape
    x = jnp.transpose(inputs, (0, 2, 3, 1)).reshape(-1, C)
    xb2 = (2.0 * x).astype(jnp.bfloat16)
    ctf = codebook.T
    ctb = ctf.astype(jnp.bfloat16)
    idx = _argmin_indices(xb2, ctb, x, ctf).reshape(-1)
    table = jnp.pad(codebook, ((0, 0), (0, GD - C)))
    q = _gather_kernel()(table, idx)[:, :C]
    ste = x + jax.lax.stop_gradient(q - x)
    return jnp.transpose(ste.reshape(B, H, W, C), (0, 3, 1, 2))
